# initial kernel scaffold (unmeasured)
import jax
import jax.numpy as jnp
from jax import lax
from jax.experimental import pallas as pl
from jax.experimental.pallas import tpu as pltpu

NX, NY, NZ = 2, 4, 4
NG = NY * NZ
D = 2048
F = 8192
FB = F // NG
HALF = D // NX


def _ring_pos(y, z):
    return y * NZ + jnp.where(y % 2 == 0, z, NZ - 1 - z)


def _ring_coords(p):
    y = p // NZ
    z = jnp.where(y % 2 == 0, p % NZ, NZ - 1 - (p % NZ))
    return y, z


def kernel(x, dy):
    m_loc = x.shape[0]

    my_x = lax.axis_index("x")
    my_y = lax.axis_index("y")
    my_z = lax.axis_index("z")
    p = _ring_pos(my_y, my_z)

    x_bf = x.astype(jnp.bfloat16)
    dy_blk = lax.dynamic_slice(dy, (0, p * FB), (m_loc, FB)).astype(jnp.bfloat16)

    def body(x_ref, dy_ref, out_ref, ag_ref, xsend_ref, xrecv_ref,
             send_sems, recv_sems, xsend_sem, xrecv_sem):
        my_x = lax.axis_index("x")
        my_y = lax.axis_index("y")
        my_z = lax.axis_index("z")
        p = _ring_pos(my_y, my_z)
        pr = (p + 1) % NG
        ry, rz = _ring_coords(pr)

        other_off = (1 - my_x) * HALF
        xsend_ref[...] = lax.dot_general(
            lax.dynamic_slice(x_ref[...], (0, other_off), (m_loc, HALF)),
            dy_ref[...],
            (((0,), (0,)), ((), ())),
            preferred_element_type=jnp.float32,
        ).astype(jnp.bfloat16)

        xrdma = pltpu.make_async_remote_copy(
            src_ref=xsend_ref,
            dst_ref=xrecv_ref,
            send_sem=xsend_sem,
            recv_sem=xrecv_sem,
            device_id=(1 - my_x, my_y, my_z),
            device_id_type=pl.DeviceIdType.MESH,
        )
        xrdma.start()

        my_off = my_x * HALF
        my_p = lax.dot_general(
            lax.dynamic_slice(x_ref[...], (0, my_off), (m_loc, HALF)),
            dy_ref[...],
            (((0,), (0,)), ((), ())),
            preferred_element_type=jnp.float32,
        )
        xrdma.wait()

        s = my_p + xrecv_ref[...].astype(jnp.float32)
        s_bf = s.astype(jnp.bfloat16)
        ag_ref[p] = s_bf
        out_ref[:, pl.ds(p * FB, FB)] = s_bf

        for h in range(NG - 1):
            sa = (p - h) % NG
            rb = (p - h - 1) % NG
            send = pltpu.make_async_remote_copy(
                src_ref=ag_ref.at[sa],
                dst_ref=ag_ref.at[sa],
                send_sem=send_sems.at[sa],
                recv_sem=recv_sems.at[sa],
                device_id=(my_x, ry, rz),
                device_id_type=pl.DeviceIdType.MESH,
            )
            send.start()
            recv = pltpu.make_async_remote_copy(
                src_ref=ag_ref.at[rb],
                dst_ref=ag_ref.at[rb],
                send_sem=send_sems.at[rb],
                recv_sem=recv_sems.at[rb],
                device_id=(my_x, ry, rz),
                device_id_type=pl.DeviceIdType.MESH,
            )
            recv.wait_recv()
            out_ref[:, pl.ds(rb * FB, FB)] = ag_ref[rb]
            send.wait_send()

    return pl.pallas_call(
        body,
        out_shape=jax.ShapeDtypeStruct((HALF, F), jnp.bfloat16),
        in_specs=[
            pl.BlockSpec(memory_space=pltpu.VMEM),
            pl.BlockSpec(memory_space=pltpu.VMEM),
        ],
        out_specs=pl.BlockSpec(memory_space=pltpu.VMEM),
        scratch_shapes=[
            pltpu.VMEM((NG, HALF, FB), jnp.bfloat16),
            pltpu.VMEM((HALF, FB), jnp.bfloat16),
            pltpu.VMEM((HALF, FB), jnp.bfloat16),
            pltpu.SemaphoreType.DMA((NG,)),
            pltpu.SemaphoreType.DMA((NG,)),
            pltpu.SemaphoreType.DMA,
            pltpu.SemaphoreType.DMA,
        ],
        compiler_params=pltpu.CompilerParams(collective_id=0),
    )(x_bf, dy_blk)


# baseline (device time: 263509 ns/iter reference)
import jax
import jax.numpy as jnp
from jax import lax
from jax.experimental import pallas as pl
from jax.experimental.pallas import tpu as pltpu

NX, NY, NZ = 2, 4, 4
NG = NY * NZ
D = 2048
F = 8192
FB = F // NG
HALF = D // NX


def _ring_pos(y, z):
    return y * NZ + jnp.where(y % 2 == 0, z, NZ - 1 - z)


def _ring_coords(p):
    y = p // NZ
    z = jnp.where(y % 2 == 0, p % NZ, NZ - 1 - (p % NZ))
    return y, z


def kernel(x, dy):
    m_loc = x.shape[0]

    my_x = lax.axis_index("x")
    my_y = lax.axis_index("y")
    my_z = lax.axis_index("z")
    p = _ring_pos(my_y, my_z)

    x_bf = x.astype(jnp.bfloat16)
    dy_blk = lax.dynamic_slice(dy, (0, p * FB), (m_loc, FB)).astype(jnp.bfloat16)

    def body(x_ref, dy_ref, out_ref, ag_ref, xsend_ref, xrecv_ref,
             send_sems, recv_sems, xsend_sem, xrecv_sem):
        my_x = lax.axis_index("x")
        my_y = lax.axis_index("y")
        my_z = lax.axis_index("z")
        p = _ring_pos(my_y, my_z)
        pr = (p + 1) % NG
        ry, rz = _ring_coords(pr)

        other_off = (1 - my_x) * HALF
        xsend_ref[...] = lax.dot_general(
            x_ref[:, pl.ds(other_off, HALF)],
            dy_ref[...],
            (((0,), (0,)), ((), ())),
            preferred_element_type=jnp.float32,
        ).astype(jnp.bfloat16)

        xrdma = pltpu.make_async_remote_copy(
            src_ref=xsend_ref,
            dst_ref=xrecv_ref,
            send_sem=xsend_sem,
            recv_sem=xrecv_sem,
            device_id=(1 - my_x, my_y, my_z),
            device_id_type=pl.DeviceIdType.MESH,
        )
        xrdma.start()

        my_off = my_x * HALF
        my_p = lax.dot_general(
            x_ref[:, pl.ds(my_off, HALF)],
            dy_ref[...],
            (((0,), (0,)), ((), ())),
            preferred_element_type=jnp.float32,
        )
        xrdma.wait()

        s = my_p + xrecv_ref[...].astype(jnp.float32)
        s_bf = s.astype(jnp.bfloat16)
        ag_ref[p] = s_bf
        out_ref[:, pl.ds(p * FB, FB)] = s_bf

        for h in range(NG - 1):
            sa = (p - h) % NG
            rb = (p - h - 1) % NG
            send = pltpu.make_async_remote_copy(
                src_ref=ag_ref.at[sa],
                dst_ref=ag_ref.at[sa],
                send_sem=send_sems.at[sa],
                recv_sem=recv_sems.at[sa],
                device_id=(my_x, ry, rz),
                device_id_type=pl.DeviceIdType.MESH,
            )
            send.start()
            recv = pltpu.make_async_remote_copy(
                src_ref=ag_ref.at[rb],
                dst_ref=ag_ref.at[rb],
                send_sem=send_sems.at[rb],
                recv_sem=recv_sems.at[rb],
                device_id=(my_x, ry, rz),
                device_id_type=pl.DeviceIdType.MESH,
            )
            recv.wait_recv()
            out_ref[:, pl.ds(rb * FB, FB)] = ag_ref[rb]
            send.wait_send()

    return pl.pallas_call(
        body,
        out_shape=jax.ShapeDtypeStruct((HALF, F), jnp.bfloat16),
        in_specs=[
            pl.BlockSpec(memory_space=pltpu.VMEM),
            pl.BlockSpec(memory_space=pltpu.VMEM),
        ],
        out_specs=pl.BlockSpec(memory_space=pltpu.VMEM),
        scratch_shapes=[
            pltpu.VMEM((NG, HALF, FB), jnp.bfloat16),
            pltpu.VMEM((HALF, FB), jnp.bfloat16),
            pltpu.VMEM((HALF, FB), jnp.bfloat16),
            pltpu.SemaphoreType.DMA((NG,)),
            pltpu.SemaphoreType.DMA((NG,)),
            pltpu.SemaphoreType.DMA,
            pltpu.SemaphoreType.DMA,
        ],
        compiler_params=pltpu.CompilerParams(
            vmem_limit_bytes=100 * 1024 * 1024,
        ),
    )(x_bf, dy_blk)


# device time: 259939 ns/iter; 1.0137x vs baseline; 1.0137x over previous
import jax
import jax.numpy as jnp
from jax import lax
from jax.experimental import pallas as pl
from jax.experimental.pallas import tpu as pltpu

NX, NY, NZ = 2, 4, 4
NG = NY * NZ
D = 2048
F = 8192
FB = F // NG
HALF = D // NX


def _ring_pos(y, z):
    return y * NZ + jnp.where(y % 2 == 0, z, NZ - 1 - z)


def _ring_coords(p):
    y = p // NZ
    z = jnp.where(y % 2 == 0, p % NZ, NZ - 1 - (p % NZ))
    return y, z


def kernel(x, dy):
    m_loc = x.shape[0]

    my_x = lax.axis_index("x")
    my_y = lax.axis_index("y")
    my_z = lax.axis_index("z")
    p = _ring_pos(my_y, my_z)

    x_bf = x.astype(jnp.bfloat16)
    dy_blk = lax.dynamic_slice(dy, (0, p * FB), (m_loc, FB)).astype(jnp.bfloat16)

    def body(x_ref, dy_ref, out_ref, ag_ref, xsend_ref, xrecv_ref,
             send_sems_r, send_sems_l, recv_sems, xsend_sem, xrecv_sem):
        my_x = lax.axis_index("x")
        my_y = lax.axis_index("y")
        my_z = lax.axis_index("z")
        p = _ring_pos(my_y, my_z)
        ry, rz = _ring_coords((p + 1) % NG)
        ly, lz = _ring_coords((p - 1) % NG)

        other_off = (1 - my_x) * HALF
        xsend_ref[...] = lax.dot_general(
            x_ref[:, pl.ds(other_off, HALF)],
            dy_ref[...],
            (((0,), (0,)), ((), ())),
            preferred_element_type=jnp.float32,
        ).astype(jnp.bfloat16)

        xrdma = pltpu.make_async_remote_copy(
            src_ref=xsend_ref,
            dst_ref=xrecv_ref,
            send_sem=xsend_sem,
            recv_sem=xrecv_sem,
            device_id=(1 - my_x, my_y, my_z),
            device_id_type=pl.DeviceIdType.MESH,
        )
        xrdma.start()

        my_off = my_x * HALF
        my_p = lax.dot_general(
            x_ref[:, pl.ds(my_off, HALF)],
            dy_ref[...],
            (((0,), (0,)), ((), ())),
            preferred_element_type=jnp.float32,
        )
        xrdma.wait()

        s = my_p + xrecv_ref[...].astype(jnp.float32)
        s_bf = s.astype(jnp.bfloat16)
        ag_ref[p] = s_bf
        out_ref[:, pl.ds(p * FB, FB)] = s_bf

        H_R, H_L = NG // 2, NG - 1 - NG // 2

        def _copy(sl, dev, ssems):
            return pltpu.make_async_remote_copy(
                src_ref=ag_ref.at[sl],
                dst_ref=ag_ref.at[sl],
                send_sem=ssems.at[sl],
                recv_sem=recv_sems.at[sl],
                device_id=dev,
                device_id_type=pl.DeviceIdType.MESH,
            )

        for h in range(H_R):
            send_r = _copy((p - h) % NG, (my_x, ry, rz), send_sems_r)
            send_r.start()
            send_l = None
            if h < H_L:
                send_l = _copy((p + h) % NG, (my_x, ly, lz), send_sems_l)
                send_l.start()
            rb_r = (p - h - 1) % NG
            _copy(rb_r, (my_x, ry, rz), send_sems_r).wait_recv()
            out_ref[:, pl.ds(rb_r * FB, FB)] = ag_ref[rb_r]
            if h < H_L:
                rb_l = (p + h + 1) % NG
                _copy(rb_l, (my_x, ly, lz), send_sems_l).wait_recv()
                out_ref[:, pl.ds(rb_l * FB, FB)] = ag_ref[rb_l]
            send_r.wait_send()
            if send_l is not None:
                send_l.wait_send()

    return pl.pallas_call(
        body,
        out_shape=jax.ShapeDtypeStruct((HALF, F), jnp.bfloat16),
        in_specs=[
            pl.BlockSpec(memory_space=pltpu.VMEM),
            pl.BlockSpec(memory_space=pltpu.VMEM),
        ],
        out_specs=pl.BlockSpec(memory_space=pltpu.VMEM),
        scratch_shapes=[
            pltpu.VMEM((NG, HALF, FB), jnp.bfloat16),
            pltpu.VMEM((HALF, FB), jnp.bfloat16),
            pltpu.VMEM((HALF, FB), jnp.bfloat16),
            pltpu.SemaphoreType.DMA((NG,)),
            pltpu.SemaphoreType.DMA((NG,)),
            pltpu.SemaphoreType.DMA((NG,)),
            pltpu.SemaphoreType.DMA,
            pltpu.SemaphoreType.DMA,
        ],
        compiler_params=pltpu.CompilerParams(
            vmem_limit_bytes=100 * 1024 * 1024,
        ),
    )(x_bf, dy_blk)


# device time: 180835 ns/iter; 1.4572x vs baseline; 1.4374x over previous
import jax
import jax.numpy as jnp
from jax import lax
from jax.experimental import pallas as pl
from jax.experimental.pallas import tpu as pltpu

NX, NY, NZ = 2, 4, 4
NG = NY * NZ
D = 2048
F = 8192
FB = F // NG
HALF = D // NX


_CYCLE = [
    (0, 0), (0, 1), (0, 2), (0, 3),
    (1, 3), (1, 2), (1, 1),
    (2, 1), (2, 2), (2, 3),
    (3, 3), (3, 2), (3, 1), (3, 0),
    (2, 0), (1, 0),
]
_YC = jnp.array([c[0] for c in _CYCLE], jnp.int32)
_ZC = jnp.array([c[1] for c in _CYCLE], jnp.int32)
_POS = jnp.zeros((NY, NZ), jnp.int32).at[
    tuple(zip(*_CYCLE))].set(jnp.arange(NG, dtype=jnp.int32))


def kernel(x, dy):
    m_loc = x.shape[0]

    my_y = lax.axis_index("y")
    my_z = lax.axis_index("z")
    p = _POS[my_y, my_z]
    pr = (p + 1) % NG
    pl_ = (p - 1) % NG
    meta = jnp.stack([p, _YC[pr], _ZC[pr], _YC[pl_], _ZC[pl_]]).astype(jnp.int32)

    x_bf = x.astype(jnp.bfloat16)
    dy_blk = lax.dynamic_slice(dy, (0, p * FB), (m_loc, FB)).astype(jnp.bfloat16)

    def body(meta_ref, x_ref, dy_ref, out_ref, ag_ref, xsend_ref, xrecv_ref,
             send_sems_r, send_sems_l, recv_sems, xsend_sem, xrecv_sem):
        my_x = lax.axis_index("x")
        my_y = lax.axis_index("y")
        my_z = lax.axis_index("z")
        p = meta_ref[0]
        ry, rz = meta_ref[1], meta_ref[2]
        ly, lz = meta_ref[3], meta_ref[4]

        other_off = (1 - my_x) * HALF
        xsend_ref[...] = lax.dot_general(
            x_ref[:, pl.ds(other_off, HALF)],
            dy_ref[...],
            (((0,), (0,)), ((), ())),
            preferred_element_type=jnp.float32,
        ).astype(jnp.bfloat16)

        xrdma = pltpu.make_async_remote_copy(
            src_ref=xsend_ref,
            dst_ref=xrecv_ref,
            send_sem=xsend_sem,
            recv_sem=xrecv_sem,
            device_id=(1 - my_x, my_y, my_z),
            device_id_type=pl.DeviceIdType.MESH,
        )
        xrdma.start()

        my_off = my_x * HALF
        my_p = lax.dot_general(
            x_ref[:, pl.ds(my_off, HALF)],
            dy_ref[...],
            (((0,), (0,)), ((), ())),
            preferred_element_type=jnp.float32,
        )
        xrdma.wait()

        s = my_p + xrecv_ref[...].astype(jnp.float32)
        s_bf = s.astype(jnp.bfloat16)
        ag_ref[p] = s_bf
        out_ref[:, pl.ds(p * FB, FB)] = s_bf

        H_R, H_L = NG // 2, NG - 1 - NG // 2

        def _copy(sl, dev, ssems):
            return pltpu.make_async_remote_copy(
                src_ref=ag_ref.at[sl],
                dst_ref=ag_ref.at[sl],
                send_sem=ssems.at[sl],
                recv_sem=recv_sems.at[sl],
                device_id=dev,
                device_id_type=pl.DeviceIdType.MESH,
            )

        for h in range(H_R):
            send_r = _copy((p - h) % NG, (my_x, ry, rz), send_sems_r)
            send_r.start()
            send_l = None
            if h < H_L:
                send_l = _copy((p + h) % NG, (my_x, ly, lz), send_sems_l)
                send_l.start()
            rb_r = (p - h - 1) % NG
            _copy(rb_r, (my_x, ry, rz), send_sems_r).wait_recv()
            out_ref[:, pl.ds(rb_r * FB, FB)] = ag_ref[rb_r]
            if h < H_L:
                rb_l = (p + h + 1) % NG
                _copy(rb_l, (my_x, ly, lz), send_sems_l).wait_recv()
                out_ref[:, pl.ds(rb_l * FB, FB)] = ag_ref[rb_l]
            send_r.wait_send()
            if send_l is not None:
                send_l.wait_send()

    return pl.pallas_call(
        body,
        out_shape=jax.ShapeDtypeStruct((HALF, F), jnp.bfloat16),
        in_specs=[
            pl.BlockSpec(memory_space=pltpu.SMEM),
            pl.BlockSpec(memory_space=pltpu.VMEM),
            pl.BlockSpec(memory_space=pltpu.VMEM),
        ],
        out_specs=pl.BlockSpec(memory_space=pltpu.VMEM),
        scratch_shapes=[
            pltpu.VMEM((NG, HALF, FB), jnp.bfloat16),
            pltpu.VMEM((HALF, FB), jnp.bfloat16),
            pltpu.VMEM((HALF, FB), jnp.bfloat16),
            pltpu.SemaphoreType.DMA((NG,)),
            pltpu.SemaphoreType.DMA((NG,)),
            pltpu.SemaphoreType.DMA((NG,)),
            pltpu.SemaphoreType.DMA,
            pltpu.SemaphoreType.DMA,
        ],
        compiler_params=pltpu.CompilerParams(
            vmem_limit_bytes=100 * 1024 * 1024,
        ),
    )(meta, x_bf, dy_blk)


# device time: 172666 ns/iter; 1.5261x vs baseline; 1.0473x over previous
import jax
import jax.numpy as jnp
from jax import lax
from jax.experimental import pallas as pl
from jax.experimental.pallas import tpu as pltpu

NX, NY, NZ = 2, 4, 4
NG = NY * NZ
D = 2048
F = 8192
FB = F // NG
HALF = D // NX


_CYCLE = [
    (0, 0), (0, 1), (0, 2), (0, 3),
    (1, 3), (1, 2), (1, 1),
    (2, 1), (2, 2), (2, 3),
    (3, 3), (3, 2), (3, 1), (3, 0),
    (2, 0), (1, 0),
]
_YC = jnp.array([c[0] for c in _CYCLE], jnp.int32)
_ZC = jnp.array([c[1] for c in _CYCLE], jnp.int32)
_POS = jnp.zeros((NY, NZ), jnp.int32).at[
    tuple(zip(*_CYCLE))].set(jnp.arange(NG, dtype=jnp.int32))


def kernel(x, dy):
    m_loc = x.shape[0]

    my_y = lax.axis_index("y")
    my_z = lax.axis_index("z")
    p = _POS[my_y, my_z]
    pr = (p + 1) % NG
    pl_ = (p - 1) % NG
    meta = jnp.stack([p, _YC[pr], _ZC[pr], _YC[pl_], _ZC[pl_]]).astype(jnp.int32)

    x_bf = x.astype(jnp.bfloat16)
    dy_blk = lax.dynamic_slice(dy, (0, p * FB), (m_loc, FB)).astype(jnp.bfloat16)

    def body(meta_ref, x_ref, dy_ref, out_ref, ag_ref, xsend_ref, xrecv_ref,
             send_sems_r, send_sems_l, recv_sems, xsend_sem, xrecv_sem):
        my_x = lax.axis_index("x")
        my_y = lax.axis_index("y")
        my_z = lax.axis_index("z")
        p = meta_ref[0]
        ry, rz = meta_ref[1], meta_ref[2]
        ly, lz = meta_ref[3], meta_ref[4]

        barrier_sem = pltpu.get_barrier_semaphore()
        for dev in [(my_x, ry, rz), (my_x, ly, lz), (1 - my_x, my_y, my_z)]:
            pl.semaphore_signal(
                barrier_sem, inc=1, device_id=dev,
                device_id_type=pl.DeviceIdType.MESH,
            )

        other_off = (1 - my_x) * HALF
        xsend_ref[...] = lax.dot_general(
            x_ref[:, pl.ds(other_off, HALF)],
            dy_ref[...],
            (((0,), (0,)), ((), ())),
            preferred_element_type=jnp.float32,
        ).astype(jnp.bfloat16)

        pl.semaphore_wait(barrier_sem, 3)
        xrdma = pltpu.make_async_remote_copy(
            src_ref=xsend_ref,
            dst_ref=xrecv_ref,
            send_sem=xsend_sem,
            recv_sem=xrecv_sem,
            device_id=(1 - my_x, my_y, my_z),
            device_id_type=pl.DeviceIdType.MESH,
        )
        xrdma.start()

        my_off = my_x * HALF
        my_p = lax.dot_general(
            x_ref[:, pl.ds(my_off, HALF)],
            dy_ref[...],
            (((0,), (0,)), ((), ())),
            preferred_element_type=jnp.float32,
        )
        xrdma.wait()

        s = my_p + xrecv_ref[...].astype(jnp.float32)
        s_bf = s.astype(jnp.bfloat16)
        ag_ref[p] = s_bf
        out_ref[:, pl.ds(p * FB, FB)] = s_bf

        H_R, H_L = NG // 2, NG - 1 - NG // 2

        def _copy(sl, dev, ssems):
            return pltpu.make_async_remote_copy(
                src_ref=ag_ref.at[sl],
                dst_ref=ag_ref.at[sl],
                send_sem=ssems.at[sl],
                recv_sem=recv_sems.at[sl],
                device_id=dev,
                device_id_type=pl.DeviceIdType.MESH,
            )

        for h in range(H_R):
            send_r = _copy((p - h) % NG, (my_x, ry, rz), send_sems_r)
            send_r.start()
            send_l = None
            if h < H_L:
                send_l = _copy((p + h) % NG, (my_x, ly, lz), send_sems_l)
                send_l.start()
            rb_r = (p - h - 1) % NG
            _copy(rb_r, (my_x, ry, rz), send_sems_r).wait_recv()
            out_ref[:, pl.ds(rb_r * FB, FB)] = ag_ref[rb_r]
            if h < H_L:
                rb_l = (p + h + 1) % NG
                _copy(rb_l, (my_x, ly, lz), send_sems_l).wait_recv()
                out_ref[:, pl.ds(rb_l * FB, FB)] = ag_ref[rb_l]
            send_r.wait_send()
            if send_l is not None:
                send_l.wait_send()

    return pl.pallas_call(
        body,
        out_shape=jax.ShapeDtypeStruct((HALF, F), jnp.bfloat16),
        in_specs=[
            pl.BlockSpec(memory_space=pltpu.SMEM),
            pl.BlockSpec(memory_space=pltpu.VMEM),
            pl.BlockSpec(memory_space=pltpu.VMEM),
        ],
        out_specs=pl.BlockSpec(memory_space=pltpu.VMEM),
        scratch_shapes=[
            pltpu.VMEM((NG, HALF, FB), jnp.bfloat16),
            pltpu.VMEM((HALF, FB), jnp.bfloat16),
            pltpu.VMEM((HALF, FB), jnp.bfloat16),
            pltpu.SemaphoreType.DMA((NG,)),
            pltpu.SemaphoreType.DMA((NG,)),
            pltpu.SemaphoreType.DMA((NG,)),
            pltpu.SemaphoreType.DMA,
            pltpu.SemaphoreType.DMA,
        ],
        compiler_params=pltpu.CompilerParams(
            vmem_limit_bytes=100 * 1024 * 1024,
            collective_id=0,
        ),
    )(meta, x_bf, dy_blk)


# device time: 171637 ns/iter; 1.5353x vs baseline; 1.0060x over previous
import jax
import jax.numpy as jnp
from jax import lax
from jax.experimental import pallas as pl
from jax.experimental.pallas import tpu as pltpu

NX, NY, NZ = 2, 4, 4
NG = NY * NZ
D = 2048
F = 8192
FB = F // NG
HALF = D // NX


_CYCLE = [
    (0, 0), (0, 1), (0, 2), (0, 3),
    (1, 3), (1, 2), (1, 1),
    (2, 1), (2, 2), (2, 3),
    (3, 3), (3, 2), (3, 1), (3, 0),
    (2, 0), (1, 0),
]
_YC = jnp.array([c[0] for c in _CYCLE], jnp.int32)
_ZC = jnp.array([c[1] for c in _CYCLE], jnp.int32)
_POS = jnp.zeros((NY, NZ), jnp.int32).at[
    tuple(zip(*_CYCLE))].set(jnp.arange(NG, dtype=jnp.int32))


def kernel(x, dy):
    m_loc = x.shape[0]

    my_y = lax.axis_index("y")
    my_z = lax.axis_index("z")
    p = _POS[my_y, my_z]
    pr = (p + 1) % NG
    pl_ = (p - 1) % NG
    meta = jnp.stack([p, _YC[pr], _ZC[pr], _YC[pl_], _ZC[pl_]]).astype(jnp.int32)

    x_bf = x.astype(jnp.bfloat16)
    dy_blk = lax.dynamic_slice(dy, (0, p * FB), (m_loc, FB)).astype(jnp.bfloat16)

    def body(meta_ref, x_ref, dy_ref, out_ref, xsend_ref, xrecv_ref,
             send_sems_r, send_sems_l, recv_sems, xsend_sem, xrecv_sem):
        my_x = lax.axis_index("x")
        my_y = lax.axis_index("y")
        my_z = lax.axis_index("z")
        p = meta_ref[0]
        ry, rz = meta_ref[1], meta_ref[2]
        ly, lz = meta_ref[3], meta_ref[4]

        barrier_sem = pltpu.get_barrier_semaphore()
        for dev in [(my_x, ry, rz), (my_x, ly, lz), (1 - my_x, my_y, my_z)]:
            pl.semaphore_signal(
                barrier_sem, inc=1, device_id=dev,
                device_id_type=pl.DeviceIdType.MESH,
            )

        other_off = (1 - my_x) * HALF
        xsend_ref[...] = lax.dot_general(
            x_ref[:, pl.ds(other_off, HALF)],
            dy_ref[...],
            (((0,), (0,)), ((), ())),
            preferred_element_type=jnp.float32,
        ).astype(jnp.bfloat16)

        pl.semaphore_wait(barrier_sem, 3)
        xrdma = pltpu.make_async_remote_copy(
            src_ref=xsend_ref,
            dst_ref=xrecv_ref,
            send_sem=xsend_sem,
            recv_sem=xrecv_sem,
            device_id=(1 - my_x, my_y, my_z),
            device_id_type=pl.DeviceIdType.MESH,
        )
        xrdma.start()

        my_off = my_x * HALF
        my_p = lax.dot_general(
            x_ref[:, pl.ds(my_off, HALF)],
            dy_ref[...],
            (((0,), (0,)), ((), ())),
            preferred_element_type=jnp.float32,
        )
        xrdma.wait()

        s = my_p + xrecv_ref[...].astype(jnp.float32)
        out_ref[:, pl.ds(p * FB, FB)] = s.astype(jnp.bfloat16)

        H_R, H_L = NG // 2, NG - 1 - NG // 2

        def _copy(sl, dev, ssems):
            return pltpu.make_async_remote_copy(
                src_ref=out_ref.at[:, pl.ds(sl * FB, FB)],
                dst_ref=out_ref.at[:, pl.ds(sl * FB, FB)],
                send_sem=ssems.at[sl],
                recv_sem=recv_sems.at[sl],
                device_id=dev,
                device_id_type=pl.DeviceIdType.MESH,
            )

        for h in range(H_R):
            send_r = _copy((p - h) % NG, (my_x, ry, rz), send_sems_r)
            send_r.start()
            send_l = None
            if h < H_L:
                send_l = _copy((p + h) % NG, (my_x, ly, lz), send_sems_l)
                send_l.start()
            rb_r = (p - h - 1) % NG
            _copy(rb_r, (my_x, ry, rz), send_sems_r).wait_recv()
            if h < H_L:
                rb_l = (p + h + 1) % NG
                _copy(rb_l, (my_x, ly, lz), send_sems_l).wait_recv()
            send_r.wait_send()
            if send_l is not None:
                send_l.wait_send()

    return pl.pallas_call(
        body,
        out_shape=jax.ShapeDtypeStruct((HALF, F), jnp.bfloat16),
        in_specs=[
            pl.BlockSpec(memory_space=pltpu.SMEM),
            pl.BlockSpec(memory_space=pltpu.VMEM),
            pl.BlockSpec(memory_space=pltpu.VMEM),
        ],
        out_specs=pl.BlockSpec(memory_space=pltpu.VMEM),
        scratch_shapes=[
            pltpu.VMEM((HALF, FB), jnp.bfloat16),
            pltpu.VMEM((HALF, FB), jnp.bfloat16),
            pltpu.SemaphoreType.DMA((NG,)),
            pltpu.SemaphoreType.DMA((NG,)),
            pltpu.SemaphoreType.DMA((NG,)),
            pltpu.SemaphoreType.DMA,
            pltpu.SemaphoreType.DMA,
        ],
        compiler_params=pltpu.CompilerParams(
            vmem_limit_bytes=100 * 1024 * 1024,
            collective_id=0,
        ),
    )(meta, x_bf, dy_blk)
